# baseline (speedup 1.0000x reference)
import functools
import math

import jax
import jax.numpy as jnp
from jax import lax
from jax.experimental import pallas as pl
from jax.experimental.pallas import tpu as pltpu
from jax.experimental.pallas import tpu_sc as plsc

D_MODEL = 128
NUM_NODES = 10000
NUM_EDGES = 320000
SCALE = math.sqrt(float(D_MODEL))

NC = 2
NS = 16
NW = NC * NS

N_PER_W = 320
N_LAST = NUM_NODES - 31 * N_PER_W
E_PER_W = NUM_EDGES // NW

CHG = 200
CHB = 200
ROUND = CHG + CHB
ROUNDS = E_PER_W // ROUND
PAIRS = ROUNDS // 2


def _scale_body(ntab_ref, etab_ref, nout_ref, eout_ref):
    nout_ref[...] = ntab_ref[...] * SCALE
    eout_ref[...] = etab_ref[...] * SCALE


def _scale_tables(node_table, edge_table):
    return pl.pallas_call(
        _scale_body,
        out_shape=(
            jax.ShapeDtypeStruct((NUM_NODES, D_MODEL), jnp.float32),
            jax.ShapeDtypeStruct((16, D_MODEL), jnp.float32),
        ),
    )(node_table, edge_table)


def _gather_body(nidx_hbm, eidx_hbm, ntab_hbm, etab_hbm,
                 nout_hbm, eout_hbm,
                 nidx_v, eidx_v, etab_v, etab_sh, g0, g1, b0, b1,
                 gsem0, gsem1, sg0, sg1, sb0, sb1):
    c = lax.axis_index("c")
    s = lax.axis_index("s")
    wid = s * NC + c

    ebase = wid * E_PER_W

    pltpu.sync_copy(eidx_hbm.at[pl.ds(ebase, E_PER_W)],
                    eidx_v.at[pl.ds(0, E_PER_W)])
    pltpu.sync_copy(etab_hbm, etab_v)

    @pl.when(s == 0)
    def _():
        pltpu.sync_copy(etab_hbm, etab_sh)

    plsc.subcore_barrier()

    @pl.when(wid < 31)
    def _():
        nbase = wid * N_PER_W
        pltpu.sync_copy(nidx_hbm.at[pl.ds(nbase, N_PER_W)], nidx_v)
        h = N_PER_W // 2
        pltpu.async_copy(ntab_hbm.at[nidx_v.at[pl.ds(0, h)]],
                         g0.at[pl.ds(0, h)], gsem0)
        pltpu.async_copy(ntab_hbm.at[nidx_v.at[pl.ds(h, h)]],
                         g1.at[pl.ds(0, h)], gsem1)
        pltpu.make_async_copy(ntab_hbm.at[nidx_v.at[pl.ds(0, h)]],
                              g0.at[pl.ds(0, h)], gsem0).wait()
        pltpu.make_async_copy(ntab_hbm.at[nidx_v.at[pl.ds(h, h)]],
                              g1.at[pl.ds(0, h)], gsem1).wait()
        pltpu.async_copy(g0.at[pl.ds(0, h)],
                         nout_hbm.at[pl.ds(nbase, h)], gsem0)
        pltpu.async_copy(g1.at[pl.ds(0, h)],
                         nout_hbm.at[pl.ds(nbase + h, h)], gsem1)
        pltpu.make_async_copy(g0.at[pl.ds(0, h)],
                              nout_hbm.at[pl.ds(nbase, h)], gsem0).wait()
        pltpu.make_async_copy(g1.at[pl.ds(0, h)],
                              nout_hbm.at[pl.ds(nbase + h, h)], gsem1).wait()

    @pl.when(wid == 31)
    def _():
        nbase = 31 * N_PER_W
        pltpu.sync_copy(nidx_hbm.at[pl.ds(nbase, N_LAST)],
                        nidx_v.at[pl.ds(0, N_LAST)])
        pltpu.async_copy(ntab_hbm.at[nidx_v.at[pl.ds(0, N_LAST)]],
                         g0.at[pl.ds(0, N_LAST)], gsem0)
        pltpu.make_async_copy(ntab_hbm.at[nidx_v.at[pl.ds(0, N_LAST)]],
                              g0.at[pl.ds(0, N_LAST)], gsem0).wait()
        pltpu.async_copy(g0.at[pl.ds(0, N_LAST)],
                         nout_hbm.at[pl.ds(nbase, N_LAST)], gsem0)
        pltpu.make_async_copy(g0.at[pl.ds(0, N_LAST)],
                              nout_hbm.at[pl.ds(nbase, N_LAST)], gsem0).wait()

    def build(buf, ioff):
        def rowgroup(g, carry):
            r0 = g * 8
            tvec = eidx_v[pl.ds(ioff + r0, 16)]
            for j in range(8):
                t = tvec[j]
                for cb in range(8):
                    sl = pl.ds(cb * 16, 16)
                    buf[r0 + j, sl] = etab_v[t, sl]
            return carry

        lax.fori_loop(0, CHB // 8, rowgroup, 0)

    def goff(r):
        return pl.multiple_of(r * ROUND, 8)

    def boff(r):
        return pl.multiple_of(r * ROUND + CHG, 8)

    def gstart(buf, sem, r):
        pltpu.async_copy(etab_sh.at[eidx_v.at[pl.ds(goff(r), CHG)]], buf, sem)

    def gwait(buf, sem):
        pltpu.make_async_copy(etab_sh.at[eidx_v.at[pl.ds(0, CHG)]], buf,
                              sem).wait()

    def sstart(buf, n, sem, ioff):
        pltpu.async_copy(buf, eout_hbm.at[pl.ds(ebase + ioff, n)], sem)

    def swait(buf, n, sem):
        pltpu.make_async_copy(buf, eout_hbm.at[pl.ds(ebase, n)], sem).wait()

    gstart(g0, gsem0, 0)

    def pair(p, carry):
        r0 = 2 * p
        r1 = r0 + 1

        @pl.when(p > 0)
        def _():
            swait(g1, CHG, sg1)
            swait(b0, CHB, sb0)

        gstart(g1, gsem1, r1)
        build(b0, boff(r0))
        gwait(g0, gsem0)
        sstart(g0, CHG, sg0, goff(r0))
        sstart(b0, CHB, sb0, boff(r0))

        @pl.when(p > 0)
        def _():
            swait(b1, CHB, sb1)

        build(b1, boff(r1))
        gwait(g1, gsem1)
        sstart(g1, CHG, sg1, goff(r1))
        sstart(b1, CHB, sb1, boff(r1))

        swait(g0, CHG, sg0)
        gstart(g0, gsem0, r0 + 2)
        return carry

    lax.fori_loop(0, PAIRS, pair, 0)

    r = ROUNDS - 1
    swait(b0, CHB, sb0)
    build(b0, boff(r))
    gwait(g0, gsem0)
    sstart(g0, CHG, sg0, goff(r))
    sstart(b0, CHB, sb0, boff(r))

    swait(g1, CHG, sg1)
    swait(b1, CHB, sb1)
    swait(g0, CHG, sg0)
    swait(b0, CHB, sb0)


def kernel(node_indices, edge_indices, edge_type_indices, node_table,
           edge_table):
    del edge_indices
    ntab_s, etab_s = _scale_tables(node_table, edge_table)

    nidx = node_indices.astype(jnp.int32)
    eidx = edge_type_indices.astype(jnp.int32)

    mesh = plsc.VectorSubcoreMesh(core_axis_name="c", subcore_axis_name="s")
    gather = functools.partial(
        pl.kernel,
        mesh=mesh,
        out_type=(
            jax.ShapeDtypeStruct((NUM_NODES, D_MODEL), jnp.float32),
            jax.ShapeDtypeStruct((NUM_EDGES, D_MODEL), jnp.float32),
        ),
        scratch_types=[
            pltpu.VMEM((N_PER_W,), jnp.int32),
            pltpu.VMEM((E_PER_W + 16,), jnp.int32),
            pltpu.VMEM((16, D_MODEL), jnp.float32),
            pltpu.VMEM_SHARED((16, D_MODEL), jnp.float32),
            pltpu.VMEM((CHG, D_MODEL), jnp.float32),
            pltpu.VMEM((CHG, D_MODEL), jnp.float32),
            pltpu.VMEM((CHB, D_MODEL), jnp.float32),
            pltpu.VMEM((CHB, D_MODEL), jnp.float32),
            pltpu.SemaphoreType.DMA,
            pltpu.SemaphoreType.DMA,
            pltpu.SemaphoreType.DMA,
            pltpu.SemaphoreType.DMA,
            pltpu.SemaphoreType.DMA,
            pltpu.SemaphoreType.DMA,
        ],
    )(_gather_body)

    nout, eout = gather(nidx, eidx, ntab_s, etab_s)
    return (nout, eout)


# speedup vs baseline: 1.4047x; 1.4047x over previous
import functools
import math

import jax
import jax.numpy as jnp
from jax import lax
from jax.experimental import pallas as pl
from jax.experimental.pallas import tpu as pltpu
from jax.experimental.pallas import tpu_sc as plsc

D_MODEL = 128
NUM_NODES = 10000
NUM_EDGES = 320000
SCALE = math.sqrt(float(D_MODEL))

NC = 2
NS = 16
NW = NC * NS

N_PER_W = 320
N_LAST = NUM_NODES - 31 * N_PER_W
E_PER_W = NUM_EDGES // NW

CHG = 280
CHB = 120
ROUND = CHG + CHB
ROUNDS = E_PER_W // ROUND
PAIRS = ROUNDS // 2


def _scale_body(ntab_ref, etab_ref, nout_ref, eout_ref):
    nout_ref[...] = ntab_ref[...] * SCALE
    eout_ref[...] = etab_ref[...] * SCALE


def _scale_tables(node_table, edge_table):
    return pl.pallas_call(
        _scale_body,
        out_shape=(
            jax.ShapeDtypeStruct((NUM_NODES, D_MODEL), jnp.float32),
            jax.ShapeDtypeStruct((16, D_MODEL), jnp.float32),
        ),
    )(node_table, edge_table)


def _gather_body(nidx_hbm, eidx_hbm, ntab_hbm, etab_hbm,
                 nout_hbm, eout_hbm,
                 nidx_v, eidx_v, etab_v, etab_sh, g0, g1, b0, b1,
                 gsem0, gsem1, sg0, sg1, sb0, sb1):
    c = lax.axis_index("c")
    s = lax.axis_index("s")
    wid = s * NC + c

    ebase = wid * E_PER_W

    pltpu.sync_copy(eidx_hbm.at[pl.ds(ebase, E_PER_W)],
                    eidx_v.at[pl.ds(0, E_PER_W)])
    pltpu.sync_copy(etab_hbm, etab_v)

    @pl.when(s == 0)
    def _():
        pltpu.sync_copy(etab_hbm, etab_sh)

    plsc.subcore_barrier()

    @pl.when(wid < 31)
    def _():
        nbase = wid * N_PER_W
        pltpu.sync_copy(nidx_hbm.at[pl.ds(nbase, N_PER_W)], nidx_v)
        h = N_PER_W // 2
        pltpu.async_copy(ntab_hbm.at[nidx_v.at[pl.ds(0, h)]],
                         g0.at[pl.ds(0, h)], gsem0)
        pltpu.async_copy(ntab_hbm.at[nidx_v.at[pl.ds(h, h)]],
                         g1.at[pl.ds(0, h)], gsem1)
        pltpu.make_async_copy(ntab_hbm.at[nidx_v.at[pl.ds(0, h)]],
                              g0.at[pl.ds(0, h)], gsem0).wait()
        pltpu.make_async_copy(ntab_hbm.at[nidx_v.at[pl.ds(h, h)]],
                              g1.at[pl.ds(0, h)], gsem1).wait()
        pltpu.async_copy(g0.at[pl.ds(0, h)],
                         nout_hbm.at[pl.ds(nbase, h)], gsem0)
        pltpu.async_copy(g1.at[pl.ds(0, h)],
                         nout_hbm.at[pl.ds(nbase + h, h)], gsem1)
        pltpu.make_async_copy(g0.at[pl.ds(0, h)],
                              nout_hbm.at[pl.ds(nbase, h)], gsem0).wait()
        pltpu.make_async_copy(g1.at[pl.ds(0, h)],
                              nout_hbm.at[pl.ds(nbase + h, h)], gsem1).wait()

    @pl.when(wid == 31)
    def _():
        nbase = 31 * N_PER_W
        pltpu.sync_copy(nidx_hbm.at[pl.ds(nbase, N_LAST)],
                        nidx_v.at[pl.ds(0, N_LAST)])
        pltpu.async_copy(ntab_hbm.at[nidx_v.at[pl.ds(0, N_LAST)]],
                         g0.at[pl.ds(0, N_LAST)], gsem0)
        pltpu.make_async_copy(ntab_hbm.at[nidx_v.at[pl.ds(0, N_LAST)]],
                              g0.at[pl.ds(0, N_LAST)], gsem0).wait()
        pltpu.async_copy(g0.at[pl.ds(0, N_LAST)],
                         nout_hbm.at[pl.ds(nbase, N_LAST)], gsem0)
        pltpu.make_async_copy(g0.at[pl.ds(0, N_LAST)],
                              nout_hbm.at[pl.ds(nbase, N_LAST)], gsem0).wait()

    def build(buf, ioff):
        def rowgroup(g, carry):
            r0 = g * 8
            tvec = eidx_v[pl.ds(ioff + r0, 16)]
            for j in range(8):
                t = tvec[j]
                for cb in range(8):
                    sl = pl.ds(cb * 16, 16)
                    buf[r0 + j, sl] = etab_v[t, sl]
            return carry

        lax.fori_loop(0, CHB // 8, rowgroup, 0)

    def goff(r):
        return pl.multiple_of(r * ROUND, 8)

    def boff(r):
        return pl.multiple_of(r * ROUND + CHG, 8)

    def gstart(buf, sem, r):
        pltpu.async_copy(etab_sh.at[eidx_v.at[pl.ds(goff(r), CHG)]], buf, sem)

    def gwait(buf, sem):
        pltpu.make_async_copy(etab_sh.at[eidx_v.at[pl.ds(0, CHG)]], buf,
                              sem).wait()

    def sstart(buf, n, sem, ioff):
        pltpu.async_copy(buf, eout_hbm.at[pl.ds(ebase + ioff, n)], sem)

    def swait(buf, n, sem):
        pltpu.make_async_copy(buf, eout_hbm.at[pl.ds(ebase, n)], sem).wait()

    gstart(g0, gsem0, 0)

    def pair(p, carry):
        r0 = 2 * p
        r1 = r0 + 1

        @pl.when(p > 0)
        def _():
            swait(g1, CHG, sg1)
            swait(b0, CHB, sb0)

        gstart(g1, gsem1, r1)
        build(b0, boff(r0))
        gwait(g0, gsem0)
        sstart(g0, CHG, sg0, goff(r0))
        sstart(b0, CHB, sb0, boff(r0))

        @pl.when(p > 0)
        def _():
            swait(b1, CHB, sb1)

        build(b1, boff(r1))
        gwait(g1, gsem1)
        sstart(g1, CHG, sg1, goff(r1))
        sstart(b1, CHB, sb1, boff(r1))

        swait(g0, CHG, sg0)
        gstart(g0, gsem0, r0 + 2)
        return carry

    lax.fori_loop(0, PAIRS, pair, 0)

    r = ROUNDS - 1
    swait(b0, CHB, sb0)
    build(b0, boff(r))
    gwait(g0, gsem0)
    sstart(g0, CHG, sg0, goff(r))
    sstart(b0, CHB, sb0, boff(r))

    swait(g1, CHG, sg1)
    swait(b1, CHB, sb1)
    swait(g0, CHG, sg0)
    swait(b0, CHB, sb0)


def kernel(node_indices, edge_indices, edge_type_indices, node_table,
           edge_table):
    del edge_indices
    ntab_s, etab_s = _scale_tables(node_table, edge_table)

    nidx = node_indices.astype(jnp.int32)
    eidx = edge_type_indices.astype(jnp.int32)

    mesh = plsc.VectorSubcoreMesh(core_axis_name="c", subcore_axis_name="s")
    gather = functools.partial(
        pl.kernel,
        mesh=mesh,
        out_type=(
            jax.ShapeDtypeStruct((NUM_NODES, D_MODEL), jnp.float32),
            jax.ShapeDtypeStruct((NUM_EDGES, D_MODEL), jnp.float32),
        ),
        scratch_types=[
            pltpu.VMEM((N_PER_W,), jnp.int32),
            pltpu.VMEM((E_PER_W + 16,), jnp.int32),
            pltpu.VMEM((16, D_MODEL), jnp.float32),
            pltpu.VMEM_SHARED((16, D_MODEL), jnp.float32),
            pltpu.VMEM((CHG, D_MODEL), jnp.float32),
            pltpu.VMEM((CHG, D_MODEL), jnp.float32),
            pltpu.VMEM((CHB, D_MODEL), jnp.float32),
            pltpu.VMEM((CHB, D_MODEL), jnp.float32),
            pltpu.SemaphoreType.DMA,
            pltpu.SemaphoreType.DMA,
            pltpu.SemaphoreType.DMA,
            pltpu.SemaphoreType.DMA,
            pltpu.SemaphoreType.DMA,
            pltpu.SemaphoreType.DMA,
        ],
    )(_gather_body)

    nout, eout = gather(nidx, eidx, ntab_s, etab_s)
    return (nout, eout)


# speedup vs baseline: 1.7465x; 1.2433x over previous
import functools
import math

import jax
import jax.numpy as jnp
from jax import lax
from jax.experimental import pallas as pl
from jax.experimental.pallas import tpu as pltpu
from jax.experimental.pallas import tpu_sc as plsc

D_MODEL = 128
NUM_NODES = 10000
NUM_EDGES = 320000
SCALE = math.sqrt(float(D_MODEL))

NC = 2
NS = 16
NW = NC * NS

N_PER_W = 320
N_LAST = NUM_NODES - 31 * N_PER_W
E_PER_W = NUM_EDGES // NW

CHG = 320
CHB = 80
ROUND = CHG + CHB
ROUNDS = E_PER_W // ROUND
PAIRS = ROUNDS // 2


def _scale_body(ntab_ref, etab_ref, nout_ref, eout_ref):
    nout_ref[...] = ntab_ref[...] * SCALE
    eout_ref[...] = etab_ref[...] * SCALE


def _scale_tables(node_table, edge_table):
    return pl.pallas_call(
        _scale_body,
        out_shape=(
            jax.ShapeDtypeStruct((NUM_NODES, D_MODEL), jnp.float32),
            jax.ShapeDtypeStruct((16, D_MODEL), jnp.float32),
        ),
    )(node_table, edge_table)


def _gather_body(nidx_hbm, eidx_hbm, ntab_hbm, etab_hbm,
                 nout_hbm, eout_hbm,
                 nidx_v, eidx_v, etab_v, etab_sh, g0, g1, b0, b1,
                 gsem0, gsem1, sg0, sg1, sb0, sb1):
    c = lax.axis_index("c")
    s = lax.axis_index("s")
    wid = s * NC + c

    ebase = wid * E_PER_W

    pltpu.sync_copy(eidx_hbm.at[pl.ds(ebase, E_PER_W)],
                    eidx_v.at[pl.ds(0, E_PER_W)])
    pltpu.sync_copy(etab_hbm, etab_v)

    @pl.when(s == 0)
    def _():
        pltpu.sync_copy(etab_hbm, etab_sh)

    plsc.subcore_barrier()

    @pl.when(wid < 31)
    def _():
        nbase = wid * N_PER_W
        pltpu.sync_copy(nidx_hbm.at[pl.ds(nbase, N_PER_W)], nidx_v)
        h = N_PER_W // 2
        pltpu.async_copy(ntab_hbm.at[nidx_v.at[pl.ds(0, h)]],
                         g0.at[pl.ds(0, h)], gsem0)
        pltpu.async_copy(ntab_hbm.at[nidx_v.at[pl.ds(h, h)]],
                         g1.at[pl.ds(0, h)], gsem1)
        pltpu.make_async_copy(ntab_hbm.at[nidx_v.at[pl.ds(0, h)]],
                              g0.at[pl.ds(0, h)], gsem0).wait()
        pltpu.make_async_copy(ntab_hbm.at[nidx_v.at[pl.ds(h, h)]],
                              g1.at[pl.ds(0, h)], gsem1).wait()
        pltpu.async_copy(g0.at[pl.ds(0, h)],
                         nout_hbm.at[pl.ds(nbase, h)], gsem0)
        pltpu.async_copy(g1.at[pl.ds(0, h)],
                         nout_hbm.at[pl.ds(nbase + h, h)], gsem1)
        pltpu.make_async_copy(g0.at[pl.ds(0, h)],
                              nout_hbm.at[pl.ds(nbase, h)], gsem0).wait()
        pltpu.make_async_copy(g1.at[pl.ds(0, h)],
                              nout_hbm.at[pl.ds(nbase + h, h)], gsem1).wait()

    @pl.when(wid == 31)
    def _():
        nbase = 31 * N_PER_W
        pltpu.sync_copy(nidx_hbm.at[pl.ds(nbase, N_LAST)],
                        nidx_v.at[pl.ds(0, N_LAST)])
        pltpu.async_copy(ntab_hbm.at[nidx_v.at[pl.ds(0, N_LAST)]],
                         g0.at[pl.ds(0, N_LAST)], gsem0)
        pltpu.make_async_copy(ntab_hbm.at[nidx_v.at[pl.ds(0, N_LAST)]],
                              g0.at[pl.ds(0, N_LAST)], gsem0).wait()
        pltpu.async_copy(g0.at[pl.ds(0, N_LAST)],
                         nout_hbm.at[pl.ds(nbase, N_LAST)], gsem0)
        pltpu.make_async_copy(g0.at[pl.ds(0, N_LAST)],
                              nout_hbm.at[pl.ds(nbase, N_LAST)], gsem0).wait()

    def build(buf, ioff):
        def rowgroup(g, carry):
            r0 = g * 8
            tvec = eidx_v[pl.ds(ioff + r0, 16)]
            for j in range(8):
                t = tvec[j]
                for cb in range(8):
                    sl = pl.ds(cb * 16, 16)
                    buf[r0 + j, sl] = etab_v[t, sl]
            return carry

        lax.fori_loop(0, CHB // 8, rowgroup, 0)

    def goff(r):
        return pl.multiple_of(r * ROUND, 8)

    def boff(r):
        return pl.multiple_of(r * ROUND + CHG, 8)

    def gstart(buf, sem, r):
        pltpu.async_copy(etab_sh.at[eidx_v.at[pl.ds(goff(r), CHG)]], buf, sem)

    def gwait(buf, sem):
        pltpu.make_async_copy(etab_sh.at[eidx_v.at[pl.ds(0, CHG)]], buf,
                              sem).wait()

    def sstart(buf, n, sem, ioff):
        pltpu.async_copy(buf, eout_hbm.at[pl.ds(ebase + ioff, n)], sem)

    def swait(buf, n, sem):
        pltpu.make_async_copy(buf, eout_hbm.at[pl.ds(ebase, n)], sem).wait()

    gstart(g0, gsem0, 0)

    def pair(p, carry):
        r0 = 2 * p
        r1 = r0 + 1

        @pl.when(p > 0)
        def _():
            swait(g1, CHG, sg1)
            swait(b0, CHB, sb0)

        gstart(g1, gsem1, r1)
        build(b0, boff(r0))
        gwait(g0, gsem0)
        sstart(g0, CHG, sg0, goff(r0))
        sstart(b0, CHB, sb0, boff(r0))

        @pl.when(p > 0)
        def _():
            swait(b1, CHB, sb1)

        build(b1, boff(r1))
        gwait(g1, gsem1)
        sstart(g1, CHG, sg1, goff(r1))
        sstart(b1, CHB, sb1, boff(r1))

        swait(g0, CHG, sg0)
        gstart(g0, gsem0, r0 + 2)
        return carry

    lax.fori_loop(0, PAIRS, pair, 0)

    r = ROUNDS - 1
    swait(b0, CHB, sb0)
    build(b0, boff(r))
    gwait(g0, gsem0)
    sstart(g0, CHG, sg0, goff(r))
    sstart(b0, CHB, sb0, boff(r))

    swait(g1, CHG, sg1)
    swait(b1, CHB, sb1)
    swait(g0, CHG, sg0)
    swait(b0, CHB, sb0)


def kernel(node_indices, edge_indices, edge_type_indices, node_table,
           edge_table):
    del edge_indices
    ntab_s, etab_s = _scale_tables(node_table, edge_table)

    nidx = node_indices.astype(jnp.int32)
    eidx = edge_type_indices.astype(jnp.int32)

    mesh = plsc.VectorSubcoreMesh(core_axis_name="c", subcore_axis_name="s")
    gather = functools.partial(
        pl.kernel,
        mesh=mesh,
        out_type=(
            jax.ShapeDtypeStruct((NUM_NODES, D_MODEL), jnp.float32),
            jax.ShapeDtypeStruct((NUM_EDGES, D_MODEL), jnp.float32),
        ),
        scratch_types=[
            pltpu.VMEM((N_PER_W,), jnp.int32),
            pltpu.VMEM((E_PER_W + 16,), jnp.int32),
            pltpu.VMEM((16, D_MODEL), jnp.float32),
            pltpu.VMEM_SHARED((16, D_MODEL), jnp.float32),
            pltpu.VMEM((CHG, D_MODEL), jnp.float32),
            pltpu.VMEM((CHG, D_MODEL), jnp.float32),
            pltpu.VMEM((CHB, D_MODEL), jnp.float32),
            pltpu.VMEM((CHB, D_MODEL), jnp.float32),
            pltpu.SemaphoreType.DMA,
            pltpu.SemaphoreType.DMA,
            pltpu.SemaphoreType.DMA,
            pltpu.SemaphoreType.DMA,
            pltpu.SemaphoreType.DMA,
            pltpu.SemaphoreType.DMA,
        ],
    )(_gather_body)

    nout, eout = gather(nidx, eidx, ntab_s, etab_s)
    return (nout, eout)


# speedup vs baseline: 1.9303x; 1.1053x over previous
import functools
import math

import jax
import jax.numpy as jnp
from jax import lax
from jax.experimental import pallas as pl
from jax.experimental.pallas import tpu as pltpu
from jax.experimental.pallas import tpu_sc as plsc

D_MODEL = 128
NUM_NODES = 10000
NUM_EDGES = 320000
SCALE = math.sqrt(float(D_MODEL))

NC = 2
NS = 16
NW = NC * NS

N_PER_W = 320
N_LAST = NUM_NODES - 31 * N_PER_W
E_PER_W = NUM_EDGES // NW

CHG = 360
CHB = 40
ROUND = CHG + CHB
ROUNDS = E_PER_W // ROUND
PAIRS = ROUNDS // 2


def _scale_body(ntab_ref, etab_ref, nout_ref, eout_ref):
    nout_ref[...] = ntab_ref[...] * SCALE
    eout_ref[...] = etab_ref[...] * SCALE


def _scale_tables(node_table, edge_table):
    return pl.pallas_call(
        _scale_body,
        out_shape=(
            jax.ShapeDtypeStruct((NUM_NODES, D_MODEL), jnp.float32),
            jax.ShapeDtypeStruct((16, D_MODEL), jnp.float32),
        ),
    )(node_table, edge_table)


def _gather_body(nidx_hbm, eidx_hbm, ntab_hbm, etab_hbm,
                 nout_hbm, eout_hbm,
                 nidx_v, eidx_v, etab_v, etab_sh, g0, g1, b0, b1,
                 gsem0, gsem1, sg0, sg1, sb0, sb1):
    c = lax.axis_index("c")
    s = lax.axis_index("s")
    wid = s * NC + c

    ebase = wid * E_PER_W

    pltpu.sync_copy(eidx_hbm.at[pl.ds(ebase, E_PER_W)],
                    eidx_v.at[pl.ds(0, E_PER_W)])
    pltpu.sync_copy(etab_hbm, etab_v)

    @pl.when(s == 0)
    def _():
        pltpu.sync_copy(etab_hbm, etab_sh)

    plsc.subcore_barrier()

    @pl.when(wid < 31)
    def _():
        nbase = wid * N_PER_W
        pltpu.sync_copy(nidx_hbm.at[pl.ds(nbase, N_PER_W)], nidx_v)
        h = N_PER_W // 2
        pltpu.async_copy(ntab_hbm.at[nidx_v.at[pl.ds(0, h)]],
                         g0.at[pl.ds(0, h)], gsem0)
        pltpu.async_copy(ntab_hbm.at[nidx_v.at[pl.ds(h, h)]],
                         g1.at[pl.ds(0, h)], gsem1)
        pltpu.make_async_copy(ntab_hbm.at[nidx_v.at[pl.ds(0, h)]],
                              g0.at[pl.ds(0, h)], gsem0).wait()
        pltpu.make_async_copy(ntab_hbm.at[nidx_v.at[pl.ds(h, h)]],
                              g1.at[pl.ds(0, h)], gsem1).wait()
        pltpu.async_copy(g0.at[pl.ds(0, h)],
                         nout_hbm.at[pl.ds(nbase, h)], gsem0)
        pltpu.async_copy(g1.at[pl.ds(0, h)],
                         nout_hbm.at[pl.ds(nbase + h, h)], gsem1)
        pltpu.make_async_copy(g0.at[pl.ds(0, h)],
                              nout_hbm.at[pl.ds(nbase, h)], gsem0).wait()
        pltpu.make_async_copy(g1.at[pl.ds(0, h)],
                              nout_hbm.at[pl.ds(nbase + h, h)], gsem1).wait()

    @pl.when(wid == 31)
    def _():
        nbase = 31 * N_PER_W
        pltpu.sync_copy(nidx_hbm.at[pl.ds(nbase, N_LAST)],
                        nidx_v.at[pl.ds(0, N_LAST)])
        pltpu.async_copy(ntab_hbm.at[nidx_v.at[pl.ds(0, N_LAST)]],
                         g0.at[pl.ds(0, N_LAST)], gsem0)
        pltpu.make_async_copy(ntab_hbm.at[nidx_v.at[pl.ds(0, N_LAST)]],
                              g0.at[pl.ds(0, N_LAST)], gsem0).wait()
        pltpu.async_copy(g0.at[pl.ds(0, N_LAST)],
                         nout_hbm.at[pl.ds(nbase, N_LAST)], gsem0)
        pltpu.make_async_copy(g0.at[pl.ds(0, N_LAST)],
                              nout_hbm.at[pl.ds(nbase, N_LAST)], gsem0).wait()

    def build(buf, ioff):
        def rowgroup(g, carry):
            r0 = g * 8
            tvec = eidx_v[pl.ds(ioff + r0, 16)]
            for j in range(8):
                t = tvec[j]
                for cb in range(8):
                    sl = pl.ds(cb * 16, 16)
                    buf[r0 + j, sl] = etab_v[t, sl]
            return carry

        lax.fori_loop(0, CHB // 8, rowgroup, 0)

    def goff(r):
        return pl.multiple_of(r * ROUND, 8)

    def boff(r):
        return pl.multiple_of(r * ROUND + CHG, 8)

    def gstart(buf, sem, r):
        pltpu.async_copy(etab_sh.at[eidx_v.at[pl.ds(goff(r), CHG)]], buf, sem)

    def gwait(buf, sem):
        pltpu.make_async_copy(etab_sh.at[eidx_v.at[pl.ds(0, CHG)]], buf,
                              sem).wait()

    def sstart(buf, n, sem, ioff):
        pltpu.async_copy(buf, eout_hbm.at[pl.ds(ebase + ioff, n)], sem)

    def swait(buf, n, sem):
        pltpu.make_async_copy(buf, eout_hbm.at[pl.ds(ebase, n)], sem).wait()

    gstart(g0, gsem0, 0)

    def pair(p, carry):
        r0 = 2 * p
        r1 = r0 + 1

        @pl.when(p > 0)
        def _():
            swait(g1, CHG, sg1)
            swait(b0, CHB, sb0)

        gstart(g1, gsem1, r1)
        build(b0, boff(r0))
        gwait(g0, gsem0)
        sstart(g0, CHG, sg0, goff(r0))
        sstart(b0, CHB, sb0, boff(r0))

        @pl.when(p > 0)
        def _():
            swait(b1, CHB, sb1)

        build(b1, boff(r1))
        gwait(g1, gsem1)
        sstart(g1, CHG, sg1, goff(r1))
        sstart(b1, CHB, sb1, boff(r1))

        swait(g0, CHG, sg0)
        gstart(g0, gsem0, r0 + 2)
        return carry

    lax.fori_loop(0, PAIRS, pair, 0)

    r = ROUNDS - 1
    swait(b0, CHB, sb0)
    build(b0, boff(r))
    gwait(g0, gsem0)
    sstart(g0, CHG, sg0, goff(r))
    sstart(b0, CHB, sb0, boff(r))

    swait(g1, CHG, sg1)
    swait(b1, CHB, sb1)
    swait(g0, CHG, sg0)
    swait(b0, CHB, sb0)


def kernel(node_indices, edge_indices, edge_type_indices, node_table,
           edge_table):
    del edge_indices
    ntab_s, etab_s = _scale_tables(node_table, edge_table)

    nidx = node_indices.astype(jnp.int32)
    eidx = edge_type_indices.astype(jnp.int32)

    mesh = plsc.VectorSubcoreMesh(core_axis_name="c", subcore_axis_name="s")
    gather = functools.partial(
        pl.kernel,
        mesh=mesh,
        out_type=(
            jax.ShapeDtypeStruct((NUM_NODES, D_MODEL), jnp.float32),
            jax.ShapeDtypeStruct((NUM_EDGES, D_MODEL), jnp.float32),
        ),
        scratch_types=[
            pltpu.VMEM((N_PER_W,), jnp.int32),
            pltpu.VMEM((E_PER_W + 16,), jnp.int32),
            pltpu.VMEM((16, D_MODEL), jnp.float32),
            pltpu.VMEM_SHARED((16, D_MODEL), jnp.float32),
            pltpu.VMEM((CHG, D_MODEL), jnp.float32),
            pltpu.VMEM((CHG, D_MODEL), jnp.float32),
            pltpu.VMEM((CHB, D_MODEL), jnp.float32),
            pltpu.VMEM((CHB, D_MODEL), jnp.float32),
            pltpu.SemaphoreType.DMA,
            pltpu.SemaphoreType.DMA,
            pltpu.SemaphoreType.DMA,
            pltpu.SemaphoreType.DMA,
            pltpu.SemaphoreType.DMA,
            pltpu.SemaphoreType.DMA,
        ],
    )(_gather_body)

    nout, eout = gather(nidx, eidx, ntab_s, etab_s)
    return (nout, eout)
